# baseline (speedup 1.0000x reference)
import functools
import math

import numpy as np

import jax
import jax.numpy as jnp
from jax import lax
from jax.experimental import pallas as pl
from jax.experimental.pallas import tpu as pltpu
from jax.experimental.pallas import tpu_sc as plsc

N = 10000
NP = 10240
E = 320000
D = 128
DE = 16
G = 64

_BNS = 1.0 / math.sqrt(1.0 + 1e-5)

_COLS_A = np.concatenate([np.arange(32 * k, 32 * k + 16) for k in range(D // 32)])
_COLS_B = _COLS_A + 16


_NC, _NS = 2, 16
_NW = _NC * _NS
_EPW = E // _NW
_B = 64
_CB = 12
_CE = _CB * _B
_NCK = 13
_NB = _NCK * _CB
_TAIL = _EPW - _NB * _B
_RPT = NP // _NS


def _sc_edge_body(h_hbm, ei_hbm, ee_hbm, out_hbm,
                  aggr, srcC, dstC,
                  src0, src1, dst0, dst1, hr0, hr1, eb0, eb1,
                  src_t, dst_t, hrow_t, ee_t, sem0, sem1):
  cid = lax.axis_index("c")
  sid = lax.axis_index("s")
  wid = cid * _NS + sid

  srcs = (src0, src1)
  dsts = (dst0, dst1)
  hrows = (hr0, hr1)
  ebs = (eb0, eb1)
  sems = (sem0, sem1)

  @plsc.parallel_loop(0, _B)
  def _(r):
    for d in range(D // 16):
      hr0[r, pl.ds(d * 16, 16)] = jnp.zeros((16,), jnp.float32)

  rbase = sid * _RPT
  for k in range(_RPT // _B):
    pltpu.sync_copy(hr0, aggr.at[pl.ds(rbase + k * _B, _B), :])

  plsc.subcore_barrier()

  ebase = wid * _EPW

  def fetch_chunk(c):
    off = ebase + c * _CE
    pltpu.sync_copy(ei_hbm.at[pl.ds(off, _CE)], srcC)
    pltpu.sync_copy(ei_hbm.at[pl.ds(E + off, _CE)], dstC)

  def stage(q, p):
    for d in range(_B // 16):
      sl = pl.ds(d * 16, 16)
      srcs[p][sl] = srcC[pl.ds(q * _B + d * 16, 16)]
      dsts[p][sl] = dstC[pl.ds(q * _B + d * 16, 16)]

  def start(off, p):
    pltpu.async_copy(h_hbm.at[srcs[p]], hrows[p], sems[p])
    pltpu.async_copy(ee_hbm.at[pl.ds(off * (D // 2), _B * (D // 2))], ebs[p],
                     sems[p])

  def finish(p):
    pltpu.make_async_copy(h_hbm.at[srcs[p]], hrows[p], sems[p]).wait()
    pltpu.make_async_copy(ee_hbm.at[pl.ds(0, _B * (D // 2))], ebs[p],
                          sems[p]).wait()

    @plsc.parallel_loop(0, _B, unroll=2)
    def _(e):
      base = pl.multiple_of(e * (D // 2), D // 2)
      for k in range(D // 32):
        w = ebs[p][pl.ds(base + k * 16, 16)]
        a = lax.bitcast_convert_type(w << 16, jnp.float32)
        b = lax.bitcast_convert_type(w & jnp.int32(-65536), jnp.float32)
        sa = pl.ds(k * 32, 16)
        sb = pl.ds(k * 32 + 16, 16)
        hrows[p][e, sa] = jnp.maximum(a + hrows[p][e, sa], 0.0)
        hrows[p][e, sb] = jnp.maximum(b + hrows[p][e, sb], 0.0)

    pltpu.sync_copy(hrows[p], aggr.at[dsts[p]], add=True)

  fetch_chunk(0)
  stage(0, 0)
  start(ebase, 0)

  def chunk_body(c, carry):
    for q in range(_CB):
      p = q % 2
      if q < _CB - 1:
        stage(q + 1, 1 - p)
        start(ebase + c * _CE + (q + 1) * _B, 1 - p)
      else:
        @pl.when(c < _NCK - 1)
        def _():
          fetch_chunk(c + 1)
          stage(0, 1 - p)
          start(ebase + (c + 1) * _CE, 1 - p)
      finish(p)
    return carry

  lax.fori_loop(0, _NCK, chunk_body, 0)

  off = ebase + _NB * _B
  pltpu.sync_copy(ei_hbm.at[pl.ds(off, _TAIL)], src_t)
  pltpu.sync_copy(ei_hbm.at[pl.ds(E + off, _TAIL)], dst_t)
  pltpu.async_copy(h_hbm.at[src_t], hrow_t, sem0).wait()
  pltpu.sync_copy(ee_hbm.at[pl.ds(off * (D // 2), _TAIL * (D // 2))], ee_t)

  @plsc.parallel_loop(0, _TAIL)
  def _(e):
    base = pl.multiple_of(e * (D // 2), D // 2)
    for k in range(D // 32):
      w = ee_t[pl.ds(base + k * 16, 16)]
      a = lax.bitcast_convert_type(w << 16, jnp.float32)
      b = lax.bitcast_convert_type(w & jnp.int32(-65536), jnp.float32)
      sa = pl.ds(k * 32, 16)
      sb = pl.ds(k * 32 + 16, 16)
      hrow_t[e, sa] = jnp.maximum(a + hrow_t[e, sa], 0.0)
      hrow_t[e, sb] = jnp.maximum(b + hrow_t[e, sb], 0.0)

  pltpu.sync_copy(hrow_t, aggr.at[dst_t], add=True)

  plsc.subcore_barrier()

  for k in range(_RPT // _B):
    pltpu.sync_copy(aggr.at[pl.ds(rbase + k * _B, _B), :],
                    out_hbm.at[cid, pl.ds(rbase + k * _B, _B), :])


_sc_edge = functools.partial(
    pl.kernel,
    out_type=jax.ShapeDtypeStruct((_NC, NP, D), jnp.float32),
    mesh=plsc.VectorSubcoreMesh(core_axis_name="c", subcore_axis_name="s"),
    scratch_types=[
        pltpu.VMEM_SHARED((NP, D), jnp.float32),
        pltpu.VMEM((_CE,), jnp.int32),
        pltpu.VMEM((_CE,), jnp.int32),
        pltpu.VMEM((_B,), jnp.int32),
        pltpu.VMEM((_B,), jnp.int32),
        pltpu.VMEM((_B,), jnp.int32),
        pltpu.VMEM((_B,), jnp.int32),
        pltpu.VMEM((_B, D), jnp.float32),
        pltpu.VMEM((_B, D), jnp.float32),
        pltpu.VMEM((_B * D // 2,), jnp.int32),
        pltpu.VMEM((_B * D // 2,), jnp.int32),
        pltpu.VMEM((_TAIL,), jnp.int32),
        pltpu.VMEM((_TAIL,), jnp.int32),
        pltpu.VMEM((_TAIL, D), jnp.float32),
        pltpu.VMEM((_TAIL * D // 2,), jnp.int32),
        pltpu.SemaphoreType.DMA,
        pltpu.SemaphoreType.DMA,
    ],
)(_sc_edge_body)


_BE = 2000
_BN = 1024


def _ee_body(ea_ref, wa0_ref, ba0_ref, wb0_ref, bb0_ref,
             wa1_ref, ba1_ref, wb1_ref, bb1_ref, o0_ref, o1_ref):
  a = ea_ref[...]

  def pack(wa_ref, ba_ref, wb_ref, bb_ref):
    va = jnp.dot(a, wa_ref[...], preferred_element_type=jnp.float32)
    va = va + ba_ref[...]
    vb = jnp.dot(a, wb_ref[...], preferred_element_type=jnp.float32)
    vb = vb + bb_ref[...]
    au = lax.bitcast_convert_type(va, jnp.uint32)
    bu = lax.bitcast_convert_type(vb, jnp.uint32)
    pk = ((au + jnp.uint32(0x8000)) >> 16) | \
         ((bu + jnp.uint32(0x8000)) & jnp.uint32(0xFFFF0000))
    return lax.bitcast_convert_type(pk, jnp.int32)

  o0_ref[...] = pack(wa0_ref, ba0_ref, wb0_ref, bb0_ref)
  o1_ref[...] = pack(wa1_ref, ba1_ref, wb1_ref, bb1_ref)


def _ee_call(ea, wa0, ba0, wb0, bb0, wa1, ba1, wb1, bb1):
  wspec = pl.BlockSpec((DE, D // 2), lambda i: (0, 0))
  bspec = pl.BlockSpec((1, D // 2), lambda i: (0, 0))
  return pl.pallas_call(
      _ee_body,
      grid=(E // _BE,),
      in_specs=[
          pl.BlockSpec((_BE, DE), lambda i: (i, 0)),
          wspec, bspec, wspec, bspec, wspec, bspec, wspec, bspec,
      ],
      out_specs=[pl.BlockSpec((_BE, D // 2), lambda i: (i, 0))] * 2,
      out_shape=[jax.ShapeDtypeStruct((E, D // 2), jnp.int32)] * 2,
  )(ea, wa0, ba0, wb0, bb0, wa1, ba1, wb1, bb1)


def _h0_body(x_ref, r_ref, o_ref):
  o_ref[...] = x_ref[...] + r_ref[...]


def _h0_call(xp, vn_row):
  return pl.pallas_call(
      _h0_body,
      grid=(NP // _BN,),
      in_specs=[
          pl.BlockSpec((_BN, D), lambda i: (i, 0)),
          pl.BlockSpec((1, D), lambda i: (0, 0)),
      ],
      out_specs=pl.BlockSpec((_BN, D), lambda i: (i, 0)),
      out_shape=jax.ShapeDtypeStruct((NP, D), jnp.float32),
  )(xp, vn_row)


def _vn_body(b_ref, h_ref, vnrow_ref, w1_ref, b1_ref, g1_ref, bt1_ref,
             w2_ref, b2_ref, g2_ref, bt2_ref, o_ref, acc_ref):
  i = pl.program_id(0)

  @pl.when(i == 0)
  def _():
    acc_ref[...] = jnp.zeros_like(acc_ref)

  bb = b_ref[...].reshape(1, _BN)
  gids = lax.broadcasted_iota(jnp.int32, (G, _BN), 0)
  oht = (gids == bb).astype(jnp.float32)
  acc_ref[...] += jnp.dot(oht, h_ref[...],
                          preferred_element_type=jnp.float32)

  @pl.when(i == NP // _BN - 1)
  def _():
    vt = acc_ref[...] + vnrow_ref[...]
    a = jnp.dot(vt, w1_ref[...], preferred_element_type=jnp.float32)
    a = a + b1_ref[...]
    a = jnp.maximum(a * _BNS * g1_ref[...] + bt1_ref[...], 0.0)
    o = jnp.dot(a, w2_ref[...], preferred_element_type=jnp.float32)
    o = o + b2_ref[...]
    o_ref[...] = jnp.maximum(o * _BNS * g2_ref[...] + bt2_ref[...], 0.0)


def _vn_call(b_row, h0, vn_row, w1, b1, g1, bt1, w2, b2, g2, bt2):
  return pl.pallas_call(
      _vn_body,
      grid=(NP // _BN,),
      in_specs=[
          pl.BlockSpec((1, 1, _BN), lambda i: (i, 0, 0)),
          pl.BlockSpec((_BN, D), lambda i: (i, 0)),
          pl.BlockSpec((1, D), lambda i: (0, 0)),
          pl.BlockSpec((D, 2 * D), lambda i: (0, 0)),
          pl.BlockSpec((1, 2 * D), lambda i: (0, 0)),
          pl.BlockSpec((1, 2 * D), lambda i: (0, 0)),
          pl.BlockSpec((1, 2 * D), lambda i: (0, 0)),
          pl.BlockSpec((2 * D, D), lambda i: (0, 0)),
          pl.BlockSpec((1, D), lambda i: (0, 0)),
          pl.BlockSpec((1, D), lambda i: (0, 0)),
          pl.BlockSpec((1, D), lambda i: (0, 0)),
      ],
      out_specs=pl.BlockSpec((G, D), lambda i: (0, 0)),
      out_shape=jax.ShapeDtypeStruct((G, D), jnp.float32),
      scratch_shapes=[pltpu.VMEM((G, D), jnp.float32)],
  )(b_row, h0, vn_row, w1, b1, g1, bt1, w2, b2, g2, bt2)


def _h1_body(h_ref, b_ref, vn_ref, o_ref):
  bcol = b_ref[...]
  gids = lax.broadcasted_iota(jnp.int32, (_BN, G), 1)
  oh = (gids == bcol).astype(jnp.float32)
  o_ref[...] = h_ref[...] + jnp.dot(oh, vn_ref[...],
                                    preferred_element_type=jnp.float32)


def _h1_call(h, b_col, vn1):
  return pl.pallas_call(
      _h1_body,
      grid=(NP // _BN,),
      in_specs=[
          pl.BlockSpec((_BN, D), lambda i: (i, 0)),
          pl.BlockSpec((_BN, 1), lambda i: (i, 0)),
          pl.BlockSpec((G, D), lambda i: (0, 0)),
      ],
      out_specs=pl.BlockSpec((_BN, D), lambda i: (i, 0)),
      out_shape=jax.ShapeDtypeStruct((NP, D), jnp.float32),
  )(h, b_col, vn1)


def _mlp_body(relu_out, h_ref, p0_ref, p1_ref, eps_ref, w1_ref, b1_ref,
              g1_ref, bt1_ref, w2_ref, b2_ref, bg_ref, bb_ref, o_ref):
  t = (1.0 + eps_ref[...]) * h_ref[...] + p0_ref[...] + p1_ref[...]
  a = jnp.dot(t, w1_ref[...], preferred_element_type=jnp.float32)
  a = a + b1_ref[...]
  a = jnp.maximum(a * _BNS * g1_ref[...] + bt1_ref[...], 0.0)
  o = jnp.dot(a, w2_ref[...], preferred_element_type=jnp.float32)
  o = o + b2_ref[...]
  o = o * _BNS * bg_ref[...] + bb_ref[...]
  if relu_out:
    o = jnp.maximum(o, 0.0)
  o_ref[...] = o


def _mlp_call(relu_out, h, p0, p1, eps, w1, b1, g1, bt1, w2, b2, bg, bb):
  return pl.pallas_call(
      functools.partial(_mlp_body, relu_out),
      grid=(NP // _BN,),
      in_specs=[
          pl.BlockSpec((_BN, D), lambda i: (i, 0)),
          pl.BlockSpec((_BN, D), lambda i: (i, 0)),
          pl.BlockSpec((_BN, D), lambda i: (i, 0)),
          pl.BlockSpec((1, 1), lambda i: (0, 0)),
          pl.BlockSpec((D, 2 * D), lambda i: (0, 0)),
          pl.BlockSpec((1, 2 * D), lambda i: (0, 0)),
          pl.BlockSpec((1, 2 * D), lambda i: (0, 0)),
          pl.BlockSpec((1, 2 * D), lambda i: (0, 0)),
          pl.BlockSpec((2 * D, D), lambda i: (0, 0)),
          pl.BlockSpec((1, D), lambda i: (0, 0)),
          pl.BlockSpec((1, D), lambda i: (0, 0)),
          pl.BlockSpec((1, D), lambda i: (0, 0)),
      ],
      out_specs=pl.BlockSpec((_BN, D), lambda i: (i, 0)),
      out_shape=jax.ShapeDtypeStruct((NP, D), jnp.float32),
  )(h, p0, p1, eps, w1, b1, g1, bt1, w2, b2, bg, bb)


def kernel(x, edge_index, edge_attr, batch, vn_table,
           l0_eps, l0_We, l0_be, l0_W1, l0_b1, l0_g1, l0_bt1, l0_W2, l0_b2,
           l0_bng, l0_bnb,
           l1_eps, l1_We, l1_be, l1_W1, l1_b1, l1_g1, l1_bt1, l1_W2, l1_b2,
           l1_bng, l1_bnb,
           vn_W1, vn_b1, vn_g1, vn_bt1, vn_W2, vn_b2, vn_g2, vn_bt2):
  r = lambda v: v.reshape(1, -1)

  xp = jnp.pad(x, ((0, NP - N), (0, 0)))
  bpad = jnp.pad(batch, (0, NP - N), constant_values=G)
  b_row = bpad.reshape(NP // _BN, 1, _BN)
  b_col = bpad.reshape(NP, 1)

  ee0, ee1 = _ee_call(edge_attr,
                      l0_We[:, _COLS_A], r(l0_be[_COLS_A]),
                      l0_We[:, _COLS_B], r(l0_be[_COLS_B]),
                      l1_We[:, _COLS_A], r(l1_be[_COLS_A]),
                      l1_We[:, _COLS_B], r(l1_be[_COLS_B]))

  h0 = _h0_call(xp, vn_table)
  ei_flat = edge_index.reshape(2 * E)
  parts0 = _sc_edge(h0, ei_flat, ee0.reshape(E * D // 2))
  vn1 = _vn_call(b_row, h0, vn_table, vn_W1, r(vn_b1), r(vn_g1), r(vn_bt1),
                 vn_W2, r(vn_b2), r(vn_g2), r(vn_bt2))
  h = _mlp_call(True, h0, parts0[0], parts0[1], l0_eps.reshape(1, 1),
                l0_W1, r(l0_b1), r(l0_g1), r(l0_bt1), l0_W2, r(l0_b2),
                r(l0_bng), r(l0_bnb))
  h1 = _h1_call(h, b_col, vn1)
  parts1 = _sc_edge(h1, ei_flat, ee1.reshape(E * D // 2))
  out = _mlp_call(False, h1, parts1[0], parts1[1], l1_eps.reshape(1, 1),
                  l1_W1, r(l1_b1), r(l1_g1), r(l1_bt1), l1_W2, r(l1_b2),
                  r(l1_bng), r(l1_bnb))
  return out[:N]


# speedup vs baseline: 1.0176x; 1.0176x over previous
import functools
import math

import numpy as np

import jax
import jax.numpy as jnp
from jax import lax
from jax.experimental import pallas as pl
from jax.experimental.pallas import tpu as pltpu
from jax.experimental.pallas import tpu_sc as plsc

N = 10000
NP = 10240
E = 320000
D = 128
DE = 16
G = 64

_BNS = 1.0 / math.sqrt(1.0 + 1e-5)

_COLS_A = np.concatenate([np.arange(32 * k, 32 * k + 16) for k in range(D // 32)])
_COLS_B = _COLS_A + 16


_NC, _NS = 2, 16
_NW = _NC * _NS
_EPW = E // _NW
_B = 64
_CB = 12
_CE = _CB * _B
_NCK = 13
_NB = _NCK * _CB
_TAIL = _EPW - _NB * _B
_RPT = NP // _NS


def _sc_edge_body(h_hbm, ei_hbm, ee_hbm, out_hbm,
                  aggr, srcC, dstC,
                  src0, src1, dst0, dst1, hr0, hr1, eb0, eb1,
                  src_t, dst_t, hrow_t, ee_t, sem0, sem1):
  cid = lax.axis_index("c")
  sid = lax.axis_index("s")
  wid = cid * _NS + sid

  srcs = (src0, src1)
  dsts = (dst0, dst1)
  hrows = (hr0, hr1)
  ebs = (eb0, eb1)
  sems = (sem0, sem1)

  @plsc.parallel_loop(0, _B)
  def _(r):
    for d in range(D // 16):
      hr0[r, pl.ds(d * 16, 16)] = jnp.zeros((16,), jnp.float32)

  rbase = sid * _RPT
  for k in range(_RPT // _B):
    pltpu.sync_copy(hr0, aggr.at[pl.ds(rbase + k * _B, _B), :])

  plsc.subcore_barrier()

  ebase = wid * _EPW

  def fetch_chunk(c):
    off = ebase + c * _CE
    pltpu.sync_copy(ei_hbm.at[pl.ds(off, _CE)], srcC)
    pltpu.sync_copy(ei_hbm.at[pl.ds(E + off, _CE)], dstC)

  def stage(q, p):
    for d in range(_B // 16):
      sl = pl.ds(d * 16, 16)
      srcs[p][sl] = srcC[pl.ds(q * _B + d * 16, 16)]
      dsts[p][sl] = dstC[pl.ds(q * _B + d * 16, 16)]

  def start(off, p):
    pltpu.async_copy(h_hbm.at[srcs[p]], hrows[p], sems[p])
    pltpu.async_copy(ee_hbm.at[pl.ds(off * (D // 2), _B * (D // 2))], ebs[p],
                     sems[p])

  def finish(p):
    pltpu.make_async_copy(h_hbm.at[srcs[p]], hrows[p], sems[p]).wait()
    pltpu.make_async_copy(ee_hbm.at[pl.ds(0, _B * (D // 2))], ebs[p],
                          sems[p]).wait()

    @plsc.parallel_loop(0, _B, unroll=2)
    def _(e):
      base = pl.multiple_of(e * (D // 2), D // 2)
      for k in range(D // 32):
        w = ebs[p][pl.ds(base + k * 16, 16)]
        a = lax.bitcast_convert_type(w << 16, jnp.float32)
        b = lax.bitcast_convert_type(w & jnp.int32(-65536), jnp.float32)
        sa = pl.ds(k * 32, 16)
        sb = pl.ds(k * 32 + 16, 16)
        hrows[p][e, sa] = jnp.maximum(a + hrows[p][e, sa], 0.0)
        hrows[p][e, sb] = jnp.maximum(b + hrows[p][e, sb], 0.0)

    pltpu.sync_copy(hrows[p], aggr.at[dsts[p]], add=True)

  fetch_chunk(0)
  stage(0, 0)
  start(ebase, 0)

  def chunk_body(c, carry):
    for q in range(_CB):
      p = q % 2
      if q < _CB - 1:
        stage(q + 1, 1 - p)
        start(ebase + c * _CE + (q + 1) * _B, 1 - p)
      else:
        @pl.when(c < _NCK - 1)
        def _():
          fetch_chunk(c + 1)
          stage(0, 1 - p)
          start(ebase + (c + 1) * _CE, 1 - p)
      finish(p)
    return carry

  lax.fori_loop(0, _NCK, chunk_body, 0)

  off = ebase + _NB * _B
  pltpu.sync_copy(ei_hbm.at[pl.ds(off, _TAIL)], src_t)
  pltpu.sync_copy(ei_hbm.at[pl.ds(E + off, _TAIL)], dst_t)
  pltpu.async_copy(h_hbm.at[src_t], hrow_t, sem0).wait()
  pltpu.sync_copy(ee_hbm.at[pl.ds(off * (D // 2), _TAIL * (D // 2))], ee_t)

  @plsc.parallel_loop(0, _TAIL)
  def _(e):
    base = pl.multiple_of(e * (D // 2), D // 2)
    for k in range(D // 32):
      w = ee_t[pl.ds(base + k * 16, 16)]
      a = lax.bitcast_convert_type(w << 16, jnp.float32)
      b = lax.bitcast_convert_type(w & jnp.int32(-65536), jnp.float32)
      sa = pl.ds(k * 32, 16)
      sb = pl.ds(k * 32 + 16, 16)
      hrow_t[e, sa] = jnp.maximum(a + hrow_t[e, sa], 0.0)
      hrow_t[e, sb] = jnp.maximum(b + hrow_t[e, sb], 0.0)

  pltpu.sync_copy(hrow_t, aggr.at[dst_t], add=True)

  plsc.subcore_barrier()

  for k in range(_RPT // _B):
    pltpu.sync_copy(aggr.at[pl.ds(rbase + k * _B, _B), :],
                    out_hbm.at[cid, pl.ds(rbase + k * _B, _B), :])


_sc_edge = functools.partial(
    pl.kernel,
    out_type=jax.ShapeDtypeStruct((_NC, NP, D), jnp.float32),
    mesh=plsc.VectorSubcoreMesh(core_axis_name="c", subcore_axis_name="s"),
    scratch_types=[
        pltpu.VMEM_SHARED((NP, D), jnp.float32),
        pltpu.VMEM((_CE,), jnp.int32),
        pltpu.VMEM((_CE,), jnp.int32),
        pltpu.VMEM((_B,), jnp.int32),
        pltpu.VMEM((_B,), jnp.int32),
        pltpu.VMEM((_B,), jnp.int32),
        pltpu.VMEM((_B,), jnp.int32),
        pltpu.VMEM((_B, D), jnp.float32),
        pltpu.VMEM((_B, D), jnp.float32),
        pltpu.VMEM((_B * D // 2,), jnp.int32),
        pltpu.VMEM((_B * D // 2,), jnp.int32),
        pltpu.VMEM((_TAIL,), jnp.int32),
        pltpu.VMEM((_TAIL,), jnp.int32),
        pltpu.VMEM((_TAIL, D), jnp.float32),
        pltpu.VMEM((_TAIL * D // 2,), jnp.int32),
        pltpu.SemaphoreType.DMA,
        pltpu.SemaphoreType.DMA,
    ],
)(_sc_edge_body)


_BE = 4000
_BN = 1024


def _ee_body(ea_ref, w_ref, b_ref, o_ref):
  a = ea_ref[...].astype(jnp.bfloat16)
  p = jnp.dot(a, w_ref[...], preferred_element_type=jnp.float32)
  p = p + b_ref[...]
  u = lax.bitcast_convert_type(p, jnp.uint32) + jnp.uint32(0x8000)
  pk = (u[:, :D // 2] >> 16) | (u[:, D // 2:] & jnp.uint32(0xFFFF0000))
  o_ref[...] = lax.bitcast_convert_type(pk, jnp.int32)


def _ee_call(ea, wcat, bcat):
  return pl.pallas_call(
      _ee_body,
      grid=(E // _BE,),
      in_specs=[
          pl.BlockSpec((_BE, DE), lambda i: (i, 0)),
          pl.BlockSpec((DE, D), lambda i: (0, 0)),
          pl.BlockSpec((1, D), lambda i: (0, 0)),
      ],
      out_specs=pl.BlockSpec((_BE, D // 2), lambda i: (i, 0)),
      out_shape=jax.ShapeDtypeStruct((E, D // 2), jnp.int32),
  )(ea, wcat, bcat)


def _h0_body(x_ref, r_ref, o_ref):
  o_ref[...] = x_ref[...] + r_ref[...]


def _h0_call(xp, vn_row):
  return pl.pallas_call(
      _h0_body,
      grid=(NP // _BN,),
      in_specs=[
          pl.BlockSpec((_BN, D), lambda i: (i, 0)),
          pl.BlockSpec((1, D), lambda i: (0, 0)),
      ],
      out_specs=pl.BlockSpec((_BN, D), lambda i: (i, 0)),
      out_shape=jax.ShapeDtypeStruct((NP, D), jnp.float32),
  )(xp, vn_row)


def _vn_body(b_ref, h_ref, vnrow_ref, w1_ref, b1_ref, g1_ref, bt1_ref,
             w2_ref, b2_ref, g2_ref, bt2_ref, o_ref, acc_ref):
  i = pl.program_id(0)

  @pl.when(i == 0)
  def _():
    acc_ref[...] = jnp.zeros_like(acc_ref)

  bb = b_ref[...].reshape(1, _BN)
  gids = lax.broadcasted_iota(jnp.int32, (G, _BN), 0)
  oht = (gids == bb).astype(jnp.float32)
  acc_ref[...] += jnp.dot(oht, h_ref[...],
                          preferred_element_type=jnp.float32)

  @pl.when(i == NP // _BN - 1)
  def _():
    vt = acc_ref[...] + vnrow_ref[...]
    a = jnp.dot(vt, w1_ref[...], preferred_element_type=jnp.float32)
    a = a + b1_ref[...]
    a = jnp.maximum(a * _BNS * g1_ref[...] + bt1_ref[...], 0.0)
    o = jnp.dot(a, w2_ref[...], preferred_element_type=jnp.float32)
    o = o + b2_ref[...]
    o_ref[...] = jnp.maximum(o * _BNS * g2_ref[...] + bt2_ref[...], 0.0)


def _vn_call(b_row, h0, vn_row, w1, b1, g1, bt1, w2, b2, g2, bt2):
  return pl.pallas_call(
      _vn_body,
      grid=(NP // _BN,),
      in_specs=[
          pl.BlockSpec((1, 1, _BN), lambda i: (i, 0, 0)),
          pl.BlockSpec((_BN, D), lambda i: (i, 0)),
          pl.BlockSpec((1, D), lambda i: (0, 0)),
          pl.BlockSpec((D, 2 * D), lambda i: (0, 0)),
          pl.BlockSpec((1, 2 * D), lambda i: (0, 0)),
          pl.BlockSpec((1, 2 * D), lambda i: (0, 0)),
          pl.BlockSpec((1, 2 * D), lambda i: (0, 0)),
          pl.BlockSpec((2 * D, D), lambda i: (0, 0)),
          pl.BlockSpec((1, D), lambda i: (0, 0)),
          pl.BlockSpec((1, D), lambda i: (0, 0)),
          pl.BlockSpec((1, D), lambda i: (0, 0)),
      ],
      out_specs=pl.BlockSpec((G, D), lambda i: (0, 0)),
      out_shape=jax.ShapeDtypeStruct((G, D), jnp.float32),
      scratch_shapes=[pltpu.VMEM((G, D), jnp.float32)],
  )(b_row, h0, vn_row, w1, b1, g1, bt1, w2, b2, g2, bt2)


def _h1_body(h_ref, b_ref, vn_ref, o_ref):
  bcol = b_ref[...]
  gids = lax.broadcasted_iota(jnp.int32, (_BN, G), 1)
  oh = (gids == bcol).astype(jnp.float32)
  o_ref[...] = h_ref[...] + jnp.dot(oh, vn_ref[...],
                                    preferred_element_type=jnp.float32)


def _h1_call(h, b_col, vn1):
  return pl.pallas_call(
      _h1_body,
      grid=(NP // _BN,),
      in_specs=[
          pl.BlockSpec((_BN, D), lambda i: (i, 0)),
          pl.BlockSpec((_BN, 1), lambda i: (i, 0)),
          pl.BlockSpec((G, D), lambda i: (0, 0)),
      ],
      out_specs=pl.BlockSpec((_BN, D), lambda i: (i, 0)),
      out_shape=jax.ShapeDtypeStruct((NP, D), jnp.float32),
  )(h, b_col, vn1)


def _mlp_body(relu_out, h_ref, p0_ref, p1_ref, eps_ref, w1_ref, b1_ref,
              g1_ref, bt1_ref, w2_ref, b2_ref, bg_ref, bb_ref, o_ref):
  t = (1.0 + eps_ref[...]) * h_ref[...] + p0_ref[...] + p1_ref[...]
  a = jnp.dot(t, w1_ref[...], preferred_element_type=jnp.float32)
  a = a + b1_ref[...]
  a = jnp.maximum(a * _BNS * g1_ref[...] + bt1_ref[...], 0.0)
  o = jnp.dot(a, w2_ref[...], preferred_element_type=jnp.float32)
  o = o + b2_ref[...]
  o = o * _BNS * bg_ref[...] + bb_ref[...]
  if relu_out:
    o = jnp.maximum(o, 0.0)
  o_ref[...] = o


def _mlp_call(relu_out, h, p0, p1, eps, w1, b1, g1, bt1, w2, b2, bg, bb):
  return pl.pallas_call(
      functools.partial(_mlp_body, relu_out),
      grid=(NP // _BN,),
      in_specs=[
          pl.BlockSpec((_BN, D), lambda i: (i, 0)),
          pl.BlockSpec((_BN, D), lambda i: (i, 0)),
          pl.BlockSpec((_BN, D), lambda i: (i, 0)),
          pl.BlockSpec((1, 1), lambda i: (0, 0)),
          pl.BlockSpec((D, 2 * D), lambda i: (0, 0)),
          pl.BlockSpec((1, 2 * D), lambda i: (0, 0)),
          pl.BlockSpec((1, 2 * D), lambda i: (0, 0)),
          pl.BlockSpec((1, 2 * D), lambda i: (0, 0)),
          pl.BlockSpec((2 * D, D), lambda i: (0, 0)),
          pl.BlockSpec((1, D), lambda i: (0, 0)),
          pl.BlockSpec((1, D), lambda i: (0, 0)),
          pl.BlockSpec((1, D), lambda i: (0, 0)),
      ],
      out_specs=pl.BlockSpec((_BN, D), lambda i: (i, 0)),
      out_shape=jax.ShapeDtypeStruct((NP, D), jnp.float32),
  )(h, p0, p1, eps, w1, b1, g1, bt1, w2, b2, bg, bb)


def kernel(x, edge_index, edge_attr, batch, vn_table,
           l0_eps, l0_We, l0_be, l0_W1, l0_b1, l0_g1, l0_bt1, l0_W2, l0_b2,
           l0_bng, l0_bnb,
           l1_eps, l1_We, l1_be, l1_W1, l1_b1, l1_g1, l1_bt1, l1_W2, l1_b2,
           l1_bng, l1_bnb,
           vn_W1, vn_b1, vn_g1, vn_bt1, vn_W2, vn_b2, vn_g2, vn_bt2):
  r = lambda v: v.reshape(1, -1)

  xp = jnp.pad(x, ((0, NP - N), (0, 0)))
  bpad = jnp.pad(batch, (0, NP - N), constant_values=G)
  b_row = bpad.reshape(NP // _BN, 1, _BN)
  b_col = bpad.reshape(NP, 1)

  w0cat = jnp.concatenate(
      [l0_We[:, _COLS_A], l0_We[:, _COLS_B]], axis=1).astype(jnp.bfloat16)
  b0cat = jnp.concatenate([l0_be[_COLS_A], l0_be[_COLS_B]]).reshape(1, D)
  w1cat = jnp.concatenate(
      [l1_We[:, _COLS_A], l1_We[:, _COLS_B]], axis=1).astype(jnp.bfloat16)
  b1cat = jnp.concatenate([l1_be[_COLS_A], l1_be[_COLS_B]]).reshape(1, D)

  ee0 = _ee_call(edge_attr, w0cat, b0cat)
  h0 = _h0_call(xp, vn_table)
  ei_flat = edge_index.reshape(2 * E)
  parts0 = _sc_edge(h0, ei_flat, ee0.reshape(E * D // 2))
  ee1 = _ee_call(edge_attr, w1cat, b1cat)
  vn1 = _vn_call(b_row, h0, vn_table, vn_W1, r(vn_b1), r(vn_g1), r(vn_bt1),
                 vn_W2, r(vn_b2), r(vn_g2), r(vn_bt2))
  h = _mlp_call(True, h0, parts0[0], parts0[1], l0_eps.reshape(1, 1),
                l0_W1, r(l0_b1), r(l0_g1), r(l0_bt1), l0_W2, r(l0_b2),
                r(l0_bng), r(l0_bnb))
  h1 = _h1_call(h, b_col, vn1)
  parts1 = _sc_edge(h1, ei_flat, ee1.reshape(E * D // 2))
  out = _mlp_call(False, h1, parts1[0], parts1[1], l1_eps.reshape(1, 1),
                  l1_W1, r(l1_b1), r(l1_g1), r(l1_bt1), l1_W2, r(l1_b2),
                  r(l1_bng), r(l1_bnb))
  return out[:N]


# speedup vs baseline: 1.2167x; 1.1957x over previous
import functools
import math

import numpy as np

import jax
import jax.numpy as jnp
from jax import lax
from jax.experimental import pallas as pl
from jax.experimental.pallas import tpu as pltpu
from jax.experimental.pallas import tpu_sc as plsc

N = 10000
NP = 10240
E = 320000
D = 128
DE = 16
G = 64

_BNS = 1.0 / math.sqrt(1.0 + 1e-5)

_COLS_A = np.concatenate([np.arange(32 * k, 32 * k + 16) for k in range(D // 32)])
_COLS_B = _COLS_A + 16


_NC, _NS = 2, 16
_NW = _NC * _NS
_EPW = E // _NW
_B = 64
_CB = 12
_CE = _CB * _B
_NCK = 13
_NB = _NCK * _CB
_TAIL = _EPW - _NB * _B
_RPT = NP // _NS


def _sc_edge_body(h_hbm, ei_hbm, ee_hbm, out_hbm,
                  aggr, srcC, dstC,
                  src0, src1, dst0, dst1, hr0, hr1, eb0, eb1,
                  src_t, dst_t, hrow_t, ee_t, sem0, sem1):
  cid = lax.axis_index("c")
  sid = lax.axis_index("s")
  wid = cid * _NS + sid

  srcs = (src0, src1)
  dsts = (dst0, dst1)
  hrows = (hr0, hr1)
  ebs = (eb0, eb1)
  sems = (sem0, sem1)

  @plsc.parallel_loop(0, _B)
  def _(r):
    for d in range(D // 16):
      hr0[r, pl.ds(d * 16, 16)] = jnp.zeros((16,), jnp.float32)

  rbase = sid * _RPT
  for k in range(_RPT // _B):
    pltpu.sync_copy(hr0, aggr.at[pl.ds(rbase + k * _B, _B), :])

  plsc.subcore_barrier()

  ebase = wid * _EPW

  def fetch_chunk(c):
    off = ebase + c * _CE
    pltpu.sync_copy(ei_hbm.at[pl.ds(off, _CE)], srcC)
    pltpu.sync_copy(ei_hbm.at[pl.ds(E + off, _CE)], dstC)

  def stage(q, p):
    for d in range(_B // 16):
      sl = pl.ds(d * 16, 16)
      srcs[p][sl] = srcC[pl.ds(q * _B + d * 16, 16)]
      dsts[p][sl] = dstC[pl.ds(q * _B + d * 16, 16)]

  def start(off, p):
    pltpu.async_copy(h_hbm.at[srcs[p]], hrows[p], sems[p])
    pltpu.async_copy(ee_hbm.at[pl.ds(off, _B), :], ebs[p], sems[p])

  def finish(p):
    pltpu.make_async_copy(h_hbm.at[srcs[p]], hrows[p], sems[p]).wait()
    pltpu.make_async_copy(ee_hbm.at[pl.ds(0, _B), :], ebs[p], sems[p]).wait()

    @plsc.parallel_loop(0, _B, unroll=2)
    def _(e):
      for k in range(D // 32):
        w = ebs[p][e, pl.ds(k * 16, 16)]
        a = lax.bitcast_convert_type(w << 16, jnp.float32)
        b = lax.bitcast_convert_type(w & jnp.int32(-65536), jnp.float32)
        sa = pl.ds(k * 32, 16)
        sb = pl.ds(k * 32 + 16, 16)
        hrows[p][e, sa] = jnp.maximum(a + hrows[p][e, sa], 0.0)
        hrows[p][e, sb] = jnp.maximum(b + hrows[p][e, sb], 0.0)

    pltpu.sync_copy(hrows[p], aggr.at[dsts[p]], add=True)

  fetch_chunk(0)
  stage(0, 0)
  start(ebase, 0)

  def chunk_body(c, carry):
    for q in range(_CB):
      p = q % 2
      if q < _CB - 1:
        stage(q + 1, 1 - p)
        start(ebase + c * _CE + (q + 1) * _B, 1 - p)
      else:
        @pl.when(c < _NCK - 1)
        def _():
          fetch_chunk(c + 1)
          stage(0, 1 - p)
          start(ebase + (c + 1) * _CE, 1 - p)
      finish(p)
    return carry

  lax.fori_loop(0, _NCK, chunk_body, 0)

  off = ebase + _NB * _B
  pltpu.sync_copy(ei_hbm.at[pl.ds(off, _TAIL)], src_t)
  pltpu.sync_copy(ei_hbm.at[pl.ds(E + off, _TAIL)], dst_t)
  pltpu.async_copy(h_hbm.at[src_t], hrow_t, sem0).wait()
  pltpu.sync_copy(ee_hbm.at[pl.ds(off, _TAIL), :], ee_t)

  @plsc.parallel_loop(0, _TAIL)
  def _(e):
    for k in range(D // 32):
      w = ee_t[e, pl.ds(k * 16, 16)]
      a = lax.bitcast_convert_type(w << 16, jnp.float32)
      b = lax.bitcast_convert_type(w & jnp.int32(-65536), jnp.float32)
      sa = pl.ds(k * 32, 16)
      sb = pl.ds(k * 32 + 16, 16)
      hrow_t[e, sa] = jnp.maximum(a + hrow_t[e, sa], 0.0)
      hrow_t[e, sb] = jnp.maximum(b + hrow_t[e, sb], 0.0)

  pltpu.sync_copy(hrow_t, aggr.at[dst_t], add=True)

  plsc.subcore_barrier()

  for k in range(_RPT // _B):
    pltpu.sync_copy(aggr.at[pl.ds(rbase + k * _B, _B), :],
                    out_hbm.at[cid, pl.ds(rbase + k * _B, _B), :])


_sc_edge = functools.partial(
    pl.kernel,
    out_type=jax.ShapeDtypeStruct((_NC, NP, D), jnp.float32),
    mesh=plsc.VectorSubcoreMesh(core_axis_name="c", subcore_axis_name="s"),
    scratch_types=[
        pltpu.VMEM_SHARED((NP, D), jnp.float32),
        pltpu.VMEM((_CE,), jnp.int32),
        pltpu.VMEM((_CE,), jnp.int32),
        pltpu.VMEM((_B,), jnp.int32),
        pltpu.VMEM((_B,), jnp.int32),
        pltpu.VMEM((_B,), jnp.int32),
        pltpu.VMEM((_B,), jnp.int32),
        pltpu.VMEM((_B, D), jnp.float32),
        pltpu.VMEM((_B, D), jnp.float32),
        pltpu.VMEM((_B, D // 2), jnp.int32),
        pltpu.VMEM((_B, D // 2), jnp.int32),
        pltpu.VMEM((_TAIL,), jnp.int32),
        pltpu.VMEM((_TAIL,), jnp.int32),
        pltpu.VMEM((_TAIL, D), jnp.float32),
        pltpu.VMEM((_TAIL, D // 2), jnp.int32),
        pltpu.SemaphoreType.DMA,
        pltpu.SemaphoreType.DMA,
    ],
)(_sc_edge_body)


_BE = 4000
_BN = 1024


def _ee_body(ea_ref, w_ref, b_ref, o_ref):
  a = ea_ref[...].astype(jnp.bfloat16)
  p = jnp.dot(a, w_ref[...], preferred_element_type=jnp.float32)
  p = p + b_ref[...]
  u = lax.bitcast_convert_type(p, jnp.uint32) + jnp.uint32(0x8000)
  pk = (u[:, :D // 2] >> 16) | (u[:, D // 2:] & jnp.uint32(0xFFFF0000))
  o_ref[...] = lax.bitcast_convert_type(pk, jnp.int32)


def _ee_call(ea, wcat, bcat):
  return pl.pallas_call(
      _ee_body,
      grid=(E // _BE,),
      in_specs=[
          pl.BlockSpec((_BE, DE), lambda i: (i, 0)),
          pl.BlockSpec((DE, D), lambda i: (0, 0)),
          pl.BlockSpec((1, D), lambda i: (0, 0)),
      ],
      out_specs=pl.BlockSpec((_BE, D // 2), lambda i: (i, 0)),
      out_shape=jax.ShapeDtypeStruct((E, D // 2), jnp.int32),
  )(ea, wcat, bcat)


def _h0_body(x_ref, r_ref, o_ref):
  o_ref[...] = x_ref[...] + r_ref[...]


def _h0_call(xp, vn_row):
  return pl.pallas_call(
      _h0_body,
      grid=(NP // _BN,),
      in_specs=[
          pl.BlockSpec((_BN, D), lambda i: (i, 0)),
          pl.BlockSpec((1, D), lambda i: (0, 0)),
      ],
      out_specs=pl.BlockSpec((_BN, D), lambda i: (i, 0)),
      out_shape=jax.ShapeDtypeStruct((NP, D), jnp.float32),
  )(xp, vn_row)


def _vn_body(b_ref, h_ref, vnrow_ref, w1_ref, b1_ref, g1_ref, bt1_ref,
             w2_ref, b2_ref, g2_ref, bt2_ref, o_ref, acc_ref):
  i = pl.program_id(0)

  @pl.when(i == 0)
  def _():
    acc_ref[...] = jnp.zeros_like(acc_ref)

  bb = b_ref[...].reshape(1, _BN)
  gids = lax.broadcasted_iota(jnp.int32, (G, _BN), 0)
  oht = (gids == bb).astype(jnp.float32)
  acc_ref[...] += jnp.dot(oht, h_ref[...],
                          preferred_element_type=jnp.float32)

  @pl.when(i == NP // _BN - 1)
  def _():
    vt = acc_ref[...] + vnrow_ref[...]
    a = jnp.dot(vt, w1_ref[...], preferred_element_type=jnp.float32)
    a = a + b1_ref[...]
    a = jnp.maximum(a * _BNS * g1_ref[...] + bt1_ref[...], 0.0)
    o = jnp.dot(a, w2_ref[...], preferred_element_type=jnp.float32)
    o = o + b2_ref[...]
    o_ref[...] = jnp.maximum(o * _BNS * g2_ref[...] + bt2_ref[...], 0.0)


def _vn_call(b_row, h0, vn_row, w1, b1, g1, bt1, w2, b2, g2, bt2):
  return pl.pallas_call(
      _vn_body,
      grid=(NP // _BN,),
      in_specs=[
          pl.BlockSpec((1, 1, _BN), lambda i: (i, 0, 0)),
          pl.BlockSpec((_BN, D), lambda i: (i, 0)),
          pl.BlockSpec((1, D), lambda i: (0, 0)),
          pl.BlockSpec((D, 2 * D), lambda i: (0, 0)),
          pl.BlockSpec((1, 2 * D), lambda i: (0, 0)),
          pl.BlockSpec((1, 2 * D), lambda i: (0, 0)),
          pl.BlockSpec((1, 2 * D), lambda i: (0, 0)),
          pl.BlockSpec((2 * D, D), lambda i: (0, 0)),
          pl.BlockSpec((1, D), lambda i: (0, 0)),
          pl.BlockSpec((1, D), lambda i: (0, 0)),
          pl.BlockSpec((1, D), lambda i: (0, 0)),
      ],
      out_specs=pl.BlockSpec((G, D), lambda i: (0, 0)),
      out_shape=jax.ShapeDtypeStruct((G, D), jnp.float32),
      scratch_shapes=[pltpu.VMEM((G, D), jnp.float32)],
  )(b_row, h0, vn_row, w1, b1, g1, bt1, w2, b2, g2, bt2)


def _h1_body(h_ref, b_ref, vn_ref, o_ref):
  bcol = b_ref[...]
  gids = lax.broadcasted_iota(jnp.int32, (_BN, G), 1)
  oh = (gids == bcol).astype(jnp.float32)
  o_ref[...] = h_ref[...] + jnp.dot(oh, vn_ref[...],
                                    preferred_element_type=jnp.float32)


def _h1_call(h, b_col, vn1):
  return pl.pallas_call(
      _h1_body,
      grid=(NP // _BN,),
      in_specs=[
          pl.BlockSpec((_BN, D), lambda i: (i, 0)),
          pl.BlockSpec((_BN, 1), lambda i: (i, 0)),
          pl.BlockSpec((G, D), lambda i: (0, 0)),
      ],
      out_specs=pl.BlockSpec((_BN, D), lambda i: (i, 0)),
      out_shape=jax.ShapeDtypeStruct((NP, D), jnp.float32),
  )(h, b_col, vn1)


def _mlp_body(relu_out, h_ref, p0_ref, p1_ref, eps_ref, w1_ref, b1_ref,
              g1_ref, bt1_ref, w2_ref, b2_ref, bg_ref, bb_ref, o_ref):
  t = (1.0 + eps_ref[...]) * h_ref[...] + p0_ref[...] + p1_ref[...]
  a = jnp.dot(t, w1_ref[...], preferred_element_type=jnp.float32)
  a = a + b1_ref[...]
  a = jnp.maximum(a * _BNS * g1_ref[...] + bt1_ref[...], 0.0)
  o = jnp.dot(a, w2_ref[...], preferred_element_type=jnp.float32)
  o = o + b2_ref[...]
  o = o * _BNS * bg_ref[...] + bb_ref[...]
  if relu_out:
    o = jnp.maximum(o, 0.0)
  o_ref[...] = o


def _mlp_call(relu_out, h, p0, p1, eps, w1, b1, g1, bt1, w2, b2, bg, bb):
  return pl.pallas_call(
      functools.partial(_mlp_body, relu_out),
      grid=(NP // _BN,),
      in_specs=[
          pl.BlockSpec((_BN, D), lambda i: (i, 0)),
          pl.BlockSpec((_BN, D), lambda i: (i, 0)),
          pl.BlockSpec((_BN, D), lambda i: (i, 0)),
          pl.BlockSpec((1, 1), lambda i: (0, 0)),
          pl.BlockSpec((D, 2 * D), lambda i: (0, 0)),
          pl.BlockSpec((1, 2 * D), lambda i: (0, 0)),
          pl.BlockSpec((1, 2 * D), lambda i: (0, 0)),
          pl.BlockSpec((1, 2 * D), lambda i: (0, 0)),
          pl.BlockSpec((2 * D, D), lambda i: (0, 0)),
          pl.BlockSpec((1, D), lambda i: (0, 0)),
          pl.BlockSpec((1, D), lambda i: (0, 0)),
          pl.BlockSpec((1, D), lambda i: (0, 0)),
      ],
      out_specs=pl.BlockSpec((_BN, D), lambda i: (i, 0)),
      out_shape=jax.ShapeDtypeStruct((NP, D), jnp.float32),
  )(h, p0, p1, eps, w1, b1, g1, bt1, w2, b2, bg, bb)


def kernel(x, edge_index, edge_attr, batch, vn_table,
           l0_eps, l0_We, l0_be, l0_W1, l0_b1, l0_g1, l0_bt1, l0_W2, l0_b2,
           l0_bng, l0_bnb,
           l1_eps, l1_We, l1_be, l1_W1, l1_b1, l1_g1, l1_bt1, l1_W2, l1_b2,
           l1_bng, l1_bnb,
           vn_W1, vn_b1, vn_g1, vn_bt1, vn_W2, vn_b2, vn_g2, vn_bt2):
  r = lambda v: v.reshape(1, -1)

  xp = jnp.pad(x, ((0, NP - N), (0, 0)))
  bpad = jnp.pad(batch, (0, NP - N), constant_values=G)
  b_row = bpad.reshape(NP // _BN, 1, _BN)
  b_col = bpad.reshape(NP, 1)

  w0cat = jnp.concatenate(
      [l0_We[:, _COLS_A], l0_We[:, _COLS_B]], axis=1).astype(jnp.bfloat16)
  b0cat = jnp.concatenate([l0_be[_COLS_A], l0_be[_COLS_B]]).reshape(1, D)
  w1cat = jnp.concatenate(
      [l1_We[:, _COLS_A], l1_We[:, _COLS_B]], axis=1).astype(jnp.bfloat16)
  b1cat = jnp.concatenate([l1_be[_COLS_A], l1_be[_COLS_B]]).reshape(1, D)

  ee0 = _ee_call(edge_attr, w0cat, b0cat)
  h0 = _h0_call(xp, vn_table)
  ei_flat = edge_index.reshape(2 * E)
  parts0 = _sc_edge(h0, ei_flat, ee0)
  ee1 = _ee_call(edge_attr, w1cat, b1cat)
  vn1 = _vn_call(b_row, h0, vn_table, vn_W1, r(vn_b1), r(vn_g1), r(vn_bt1),
                 vn_W2, r(vn_b2), r(vn_g2), r(vn_bt2))
  h = _mlp_call(True, h0, parts0[0], parts0[1], l0_eps.reshape(1, 1),
                l0_W1, r(l0_b1), r(l0_g1), r(l0_bt1), l0_W2, r(l0_b2),
                r(l0_bng), r(l0_bnb))
  h1 = _h1_call(h, b_col, vn1)
  parts1 = _sc_edge(h1, ei_flat, ee1)
  out = _mlp_call(False, h1, parts1[0], parts1[1], l1_eps.reshape(1, 1),
                  l1_W1, r(l1_b1), r(l1_g1), r(l1_bt1), l1_W2, r(l1_b2),
                  r(l1_bng), r(l1_bnb))
  return out[:N]
